# baseline (speedup 1.0000x reference)
import jax
import jax.numpy as jnp
from jax import lax
from jax.experimental import pallas as pl
from jax.experimental.pallas import tpu as pltpu, tpu_sc as plsc

N = 10000
D = 128
E = 320000
GAMMA = 0.5

NC = 2
NS = 16
L = 16

HALF = D // NC
EPT = E // NS
CH = 80
NCHUNK = EPT // CH
UN = 8
NB = 4
NP = (NCHUNK - 2) // UN
RCH = 80
NRCH = N // RCH
RITER = -(-NRCH // NS)


def _body(h2_hbm, src_hbm, dst_hbm, e_hbm, out0, out1,
          srcv, dstv, ev, grow, srow, fidx_v, fbuf_v, hbuf_bf, h_sp, acc,
          gsem, ssem, lsem, dsem):
    c = lax.axis_index("c")
    s = lax.axis_index("s")
    lane = lax.iota(jnp.int32, L)
    ebase = s * EPT

    def init_chunk(i, _):
        cid = s + i * NS

        @pl.when(cid < NRCH)
        def _():
            base_r = cid * RCH
            for v in range(RCH // L):
                fidx_v[pl.ds(v * L, L)] = (base_r + v * L + lane) * 2 + c
            pltpu.async_copy(h2_hbm.at[fidx_v], fbuf_v, gsem.at[0]).wait()

            @plsc.parallel_loop(0, RCH, unroll=2)
            def _(j):
                for q in range(HALF // (2 * L)):
                    a = fbuf_v[j, pl.ds(q * 2 * L, L)]
                    b = fbuf_v[j, pl.ds(q * 2 * L + L, L)]
                    hbuf_bf[j, pl.ds(q * 2 * L, 2 * L)] = plsc.pack(
                        a, b, format=plsc.PackFormat.INTERLEAVED)
            pltpu.sync_copy(hbuf_bf, h_sp.at[pl.ds(base_r, RCH)])

            @plsc.parallel_loop(0, RCH, unroll=2)
            def _(j):
                for q in range(HALF // L):
                    sl = pl.ds(q * L, L)
                    fbuf_v[j, sl] = fbuf_v[j, sl] * (-GAMMA)
            pltpu.sync_copy(fbuf_v, acc.at[pl.ds(base_r, RCH)])
        return 0
    lax.fori_loop(0, RITER, init_chunk, 0)
    plsc.subcore_barrier()

    def issue_srce(i, k):
        off = ebase + i * CH
        pltpu.async_copy(src_hbm.at[pl.ds(off, CH)], srcv.at[k], lsem.at[k])
        pltpu.async_copy(e_hbm.at[pl.ds(off, CH)], ev.at[k], lsem.at[k])

    def wait_srce(i, k):
        off = ebase + i * CH
        pltpu.make_async_copy(src_hbm.at[pl.ds(off, CH)], srcv.at[k],
                              lsem.at[k]).wait()
        pltpu.make_async_copy(e_hbm.at[pl.ds(off, CH)], ev.at[k],
                              lsem.at[k]).wait()

    def issue_dst(i, k):
        off = ebase + i * CH
        pltpu.async_copy(dst_hbm.at[pl.ds(off, CH)], dstv.at[k], dsem.at[k])

    def wait_dst(i, k):
        off = ebase + i * CH
        pltpu.make_async_copy(dst_hbm.at[pl.ds(off, CH)], dstv.at[k],
                              dsem.at[k]).wait()

    def issue_gather(k, b):
        pltpu.async_copy(h_sp.at[srcv.at[k]], grow.at[b], gsem.at[b])

    def wait_gather(k, b):
        pltpu.make_async_copy(h_sp.at[srcv.at[k]], grow.at[b],
                              gsem.at[b]).wait()

    def issue_scatter(k, b):
        pltpu.async_copy(srow.at[b], acc.at[dstv.at[k]], ssem.at[b],
                         add=True)

    def wait_scatter(k, b):
        pltpu.make_async_copy(srow.at[b], acc.at[dstv.at[k]],
                              ssem.at[b]).wait()

    def mul_chunk(k, b):
        @plsc.parallel_loop(0, CH, unroll=8)
        def _(j):
            e16 = ev[k, pl.ds(j, L)]
            eb = lax.gather(
                e16, jnp.zeros((L, 1), jnp.int32),
                lax.GatherDimensionNumbers(
                    offset_dims=(), collapsed_slice_dims=(0,),
                    start_index_map=(0,)),
                (1,), mode=lax.GatherScatterMode.PROMISE_IN_BOUNDS)
            for q in range(HALF // (2 * L)):
                v32 = grow[b, j, pl.ds(q * 2 * L, 2 * L)]
                a, b2 = plsc.unpack(v32,
                                    format=plsc.PackFormat.INTERLEAVED)
                srow[b, j, pl.ds(q * 2 * L, L)] = a * eb
                srow[b, j, pl.ds(q * 2 * L + L, L)] = b2 * eb

    def chunk_body(i, k):
        b = k % NB
        wait_gather(k, b)

        @pl.when(i >= NB)
        def _():
            wait_scatter((k + NB) % UN, b)
        mul_chunk(k, b)
        wait_dst(i, k)
        issue_scatter(k, b)

        @pl.when(i + NB < NCHUNK)
        def _():
            wait_srce(i + NB, (k + NB) % UN)
            issue_gather((k + NB) % UN, b)
            issue_dst(i + NB, (k + NB) % UN)

        @pl.when(i + 6 < NCHUNK)
        def _():
            issue_srce(i + 6, (k + 6) % UN)

    for i0 in range(6):
        issue_srce(i0, i0)
    for i0 in range(NB):
        issue_dst(i0, i0)
    for i0 in range(NB):
        wait_srce(i0, i0)
        issue_gather(i0, i0)

    def pipe_step(p, _):
        for k in range(UN):
            chunk_body(p * UN + k, k)
        return 0
    lax.fori_loop(0, NP, pipe_step, 0)
    chunk_body(NCHUNK - 2, (NCHUNK - 2) % UN)
    chunk_body(NCHUNK - 1, (NCHUNK - 1) % UN)
    for i0 in range(NCHUNK - NB, NCHUNK):
        wait_scatter(i0 % UN, i0 % NB)
    plsc.subcore_barrier()

    def out_chunk(i, _):
        cid = s + i * NS

        @pl.when(cid < NRCH)
        def _():
            base_r = cid * RCH
            pltpu.sync_copy(acc.at[pl.ds(base_r, RCH)], fbuf_v)

            @pl.when(c == 0)
            def _():
                pltpu.sync_copy(fbuf_v, out0.at[pl.ds(base_r, RCH)])

            @pl.when(c == 1)
            def _():
                pltpu.sync_copy(fbuf_v, out1.at[pl.ds(base_r, RCH)])
        return 0
    lax.fori_loop(0, RITER, out_chunk, 0)


@jax.jit
def _run(h2, src, dst, e):
    mesh = plsc.VectorSubcoreMesh(core_axis_name="c", subcore_axis_name="s",
                                  num_cores=NC, num_subcores=NS)
    f = pl.kernel(
        _body,
        out_type=(jax.ShapeDtypeStruct((N, HALF), jnp.float32),
                  jax.ShapeDtypeStruct((N, HALF), jnp.float32)),
        mesh=mesh,
        scratch_types=[
            pltpu.VMEM((UN, CH), jnp.int32),
            pltpu.VMEM((UN, CH), jnp.int32),
            pltpu.VMEM((UN, CH), jnp.float32),
            pltpu.VMEM((NB, CH, HALF), jnp.bfloat16),
            pltpu.VMEM((NB, CH, HALF), jnp.float32),
            pltpu.VMEM((RCH,), jnp.int32),
            pltpu.VMEM((RCH, HALF), jnp.float32),
            pltpu.VMEM((RCH, HALF), jnp.bfloat16),
            pltpu.VMEM_SHARED((N, HALF), jnp.bfloat16),
            pltpu.VMEM_SHARED((N, HALF), jnp.float32),
            pltpu.SemaphoreType.DMA((NB,)),
            pltpu.SemaphoreType.DMA((NB,)),
            pltpu.SemaphoreType.DMA((UN,)),
            pltpu.SemaphoreType.DMA((UN,)),
        ],
        compiler_params=pltpu.CompilerParams(needs_layout_passes=False,
                                             use_tc_tiling_on_sc=False),
    )
    return f(h2, src, dst, e)


def kernel(t, x, edge_index):
    h2 = x[: N * D].reshape(N * NC, HALF)
    e = x[N * D:]
    src = edge_index[0].astype(jnp.int32)
    dst = edge_index[1].astype(jnp.int32)
    o0, o1 = _run(h2, src, dst, e)
    h_new = jnp.concatenate([o0, o1], axis=1)
    return jnp.concatenate([h_new.reshape(-1), jnp.zeros((E,), x.dtype)])


# speedup vs baseline: 1.1609x; 1.1609x over previous
import jax
import jax.numpy as jnp
from jax import lax
from jax.experimental import pallas as pl
from jax.experimental.pallas import tpu as pltpu, tpu_sc as plsc

N = 10000
D = 128
E = 320000
GAMMA = 0.5

NC = 2
NS = 16
L = 16

HALF = D // NC
EPT = E // NS
CH = 80
NCHUNK = EPT // CH
UN = 8
NB = 4
NP = (NCHUNK - 2) // UN
RCH = 80
NRCH = N // RCH
RITER = -(-NRCH // NS)


def _body(h2_hbm, src_hbm, dst_hbm, e_hbm, out0, out1,
          srcv, dstv, ev, grow, srow, fidx_v, fbuf_v, h_sp, acc,
          gsem, ssem, lsem, dsem):
    c = lax.axis_index("c")
    s = lax.axis_index("s")
    lane = lax.iota(jnp.int32, L)
    ebase = s * EPT

    def init_chunk(i, _):
        cid = s + i * NS

        @pl.when(cid < NRCH)
        def _():
            base_r = cid * RCH
            for v in range(RCH // L):
                fidx_v[pl.ds(v * L, L)] = (base_r + v * L + lane) * 2 + c
            pltpu.async_copy(h2_hbm.at[fidx_v], fbuf_v, gsem.at[0]).wait()
            pltpu.sync_copy(fbuf_v, h_sp.at[pl.ds(base_r, RCH)])

            @plsc.parallel_loop(0, RCH, unroll=2)
            def _(j):
                for q in range(HALF // L):
                    sl = pl.ds(q * L, L)
                    fbuf_v[j, sl] = fbuf_v[j, sl] * (-GAMMA)
            pltpu.sync_copy(fbuf_v, acc.at[pl.ds(base_r, RCH)])
        return 0
    lax.fori_loop(0, RITER, init_chunk, 0)
    plsc.subcore_barrier()

    def issue_srce(i, k):
        off = ebase + i * CH
        pltpu.async_copy(src_hbm.at[pl.ds(off, CH)], srcv.at[k], lsem.at[k])
        pltpu.async_copy(e_hbm.at[pl.ds(off, CH)], ev.at[k], lsem.at[k])

    def wait_srce(i, k):
        off = ebase + i * CH
        pltpu.make_async_copy(src_hbm.at[pl.ds(off, CH)], srcv.at[k],
                              lsem.at[k]).wait()
        pltpu.make_async_copy(e_hbm.at[pl.ds(off, CH)], ev.at[k],
                              lsem.at[k]).wait()

    def issue_dst(i, k):
        off = ebase + i * CH
        pltpu.async_copy(dst_hbm.at[pl.ds(off, CH)], dstv.at[k], dsem.at[k])

    def wait_dst(i, k):
        off = ebase + i * CH
        pltpu.make_async_copy(dst_hbm.at[pl.ds(off, CH)], dstv.at[k],
                              dsem.at[k]).wait()

    def issue_gather(k, b):
        pltpu.async_copy(h_sp.at[srcv.at[k]], grow.at[b], gsem.at[b])

    def wait_gather(k, b):
        pltpu.make_async_copy(h_sp.at[srcv.at[k]], grow.at[b],
                              gsem.at[b]).wait()

    def issue_scatter(k, b):
        pltpu.async_copy(srow.at[b], acc.at[dstv.at[k]], ssem.at[b],
                         add=True)

    def wait_scatter(k, b):
        pltpu.make_async_copy(srow.at[b], acc.at[dstv.at[k]],
                              ssem.at[b]).wait()

    def mul_chunk(k, b):
        @plsc.parallel_loop(0, CH, unroll=8)
        def _(j):
            e16 = ev[k, pl.ds(j, L)]
            eb = lax.gather(
                e16, jnp.zeros((L, 1), jnp.int32),
                lax.GatherDimensionNumbers(
                    offset_dims=(), collapsed_slice_dims=(0,),
                    start_index_map=(0,)),
                (1,), mode=lax.GatherScatterMode.PROMISE_IN_BOUNDS)
            for q in range(HALF // L):
                sl = pl.ds(q * L, L)
                srow[b, j, sl] = grow[b, j, sl] * eb

    def chunk_body(i, k):
        b = k % NB
        wait_gather(k, b)

        @pl.when(i >= NB)
        def _():
            wait_scatter((k + NB) % UN, b)
        mul_chunk(k, b)
        wait_dst(i, k)
        issue_scatter(k, b)

        @pl.when(i + NB < NCHUNK)
        def _():
            wait_srce(i + NB, (k + NB) % UN)
            issue_gather((k + NB) % UN, b)
            issue_dst(i + NB, (k + NB) % UN)

        @pl.when(i + 6 < NCHUNK)
        def _():
            issue_srce(i + 6, (k + 6) % UN)

    for i0 in range(6):
        issue_srce(i0, i0)
    for i0 in range(NB):
        issue_dst(i0, i0)
    for i0 in range(NB):
        wait_srce(i0, i0)
        issue_gather(i0, i0)

    def pipe_step(p, _):
        for k in range(UN):
            chunk_body(p * UN + k, k)
        return 0
    lax.fori_loop(0, NP, pipe_step, 0)
    chunk_body(NCHUNK - 2, (NCHUNK - 2) % UN)
    chunk_body(NCHUNK - 1, (NCHUNK - 1) % UN)
    for i0 in range(NCHUNK - NB, NCHUNK):
        wait_scatter(i0 % UN, i0 % NB)
    plsc.subcore_barrier()

    def out_chunk(i, _):
        cid = s + i * NS

        @pl.when(cid < NRCH)
        def _():
            base_r = cid * RCH
            pltpu.sync_copy(acc.at[pl.ds(base_r, RCH)], fbuf_v)

            @pl.when(c == 0)
            def _():
                pltpu.sync_copy(fbuf_v, out0.at[pl.ds(base_r, RCH)])

            @pl.when(c == 1)
            def _():
                pltpu.sync_copy(fbuf_v, out1.at[pl.ds(base_r, RCH)])
        return 0
    lax.fori_loop(0, RITER, out_chunk, 0)


@jax.jit
def _run(h2, src, dst, e):
    mesh = plsc.VectorSubcoreMesh(core_axis_name="c", subcore_axis_name="s",
                                  num_cores=NC, num_subcores=NS)
    f = pl.kernel(
        _body,
        out_type=(jax.ShapeDtypeStruct((N, HALF), jnp.float32),
                  jax.ShapeDtypeStruct((N, HALF), jnp.float32)),
        mesh=mesh,
        scratch_types=[
            pltpu.VMEM((UN, CH), jnp.int32),
            pltpu.VMEM((UN, CH), jnp.int32),
            pltpu.VMEM((UN, CH), jnp.float32),
            pltpu.VMEM((NB, CH, HALF), jnp.float32),
            pltpu.VMEM((NB, CH, HALF), jnp.float32),
            pltpu.VMEM((RCH,), jnp.int32),
            pltpu.VMEM((RCH, HALF), jnp.float32),
            pltpu.VMEM_SHARED((N, HALF), jnp.float32),
            pltpu.VMEM_SHARED((N, HALF), jnp.float32),
            pltpu.SemaphoreType.DMA((NB,)),
            pltpu.SemaphoreType.DMA((NB,)),
            pltpu.SemaphoreType.DMA((UN,)),
            pltpu.SemaphoreType.DMA((UN,)),
        ],
        compiler_params=pltpu.CompilerParams(needs_layout_passes=False,
                                             use_tc_tiling_on_sc=False),
    )
    return f(h2, src, dst, e)


def kernel(t, x, edge_index):
    h2 = x[: N * D].reshape(N * NC, HALF)
    e = x[N * D:]
    src = edge_index[0].astype(jnp.int32)
    dst = edge_index[1].astype(jnp.int32)
    o0, o1 = _run(h2, src, dst, e)
    h_new = jnp.concatenate([o0, o1], axis=1)
    return jnp.concatenate([h_new.reshape(-1), jnp.zeros((E,), x.dtype)])
